# baseline (speedup 1.0000x reference)
import jax
import jax.numpy as jnp
from jax import lax
from jax.experimental import pallas as pl
from jax.experimental.pallas import tpu as pltpu
from jax.experimental.pallas import tpu_sc as plsc

_T = 0.0025
_M = 0.6
_K = 16
_N = 256
_L = 16
_NC = 1
_NS = 16
_NW = _NC * _NS
_NTC = 0
_NSC = _N - _NTC
_RPW = _NSC // _NW
_NCHUNK = _N // _L


def _tc_dist_body(yi_ref, yit_ref, comb_ref, e2_ref):
    yi = yi_ref[...]
    yit = yit_ref[...]
    yin = yi * lax.rsqrt(jnp.sum(yi * yi, axis=1, keepdims=True) + 1e-12)
    yitn = yit * lax.rsqrt(jnp.sum(yit * yit, axis=1, keepdims=True) + 1e-12)
    g = lax.dot_general(yin, yin, (((1,), (1,)), ((), ())),
                        preferred_element_type=jnp.float32)
    c = lax.dot_general(yitn, yin, (((1,), (1,)), ((), ())),
                        preferred_element_type=jnp.float32)
    dii = 0.5 * jnp.sqrt(jnp.maximum(2.0 - 2.0 * g, 0.0) + 1e-12)
    dt = 0.5 * jnp.sqrt(jnp.maximum(2.0 - 2.0 * c, 0.0) + 1e-12)
    diag = (lax.broadcasted_iota(jnp.int32, (_N, _N), 0)
            == lax.broadcasted_iota(jnp.int32, (_N, _N), 1))
    dii_s = jnp.where(diag, 2.0, dii)
    comb_ref[:, : _N] = dii_s
    comb_ref[:, _N:] = dt
    d1 = jnp.min(dii_s, axis=1)
    diag_t = jnp.sum(jnp.where(diag, dt, 0.0), axis=1)
    e2 = jnp.sum(jnp.maximum(diag_t + _M - d1, 0.0))
    e2_ref[...] = jnp.reshape(e2 - _T * (_K * _N), (1, 1))


def _sc_body(comb_hbm, out_hbm, comb_v, acc_v):
    cid = lax.axis_index("c")
    sid = lax.axis_index("s")
    wid = sid * _NC + cid
    base = wid * _RPW
    pltpu.sync_copy(comb_hbm.at[pl.ds(base, _RPW)], comb_v)
    total = jnp.zeros((_L,), jnp.float32)
    for r in range(_RPW):
        bk = bv = None
        for c in range(_NCHUNK):
            key = comb_v[r, pl.ds(c * _L, _L)]
            val = comb_v[r, pl.ds(_N + c * _L, _L)]
            if c == 0:
                bk, bv = plsc.sort_key_val(key, val)
            else:
                ck, cv = plsc.sort_key_val(key, val, descending=True)
                keep = bk <= ck
                lk = jnp.where(keep, bk, ck)
                lv = jnp.where(keep, bv, cv)
                bk, bv = plsc.sort_key_val(lk, lv)
        diff = bk - bv
        total = total + diff * diff
    acc_v[...] = total
    pltpu.sync_copy(acc_v, out_hbm.at[wid])


_sc_knn = pl.kernel(
    _sc_body,
    out_type=jax.ShapeDtypeStruct((_NW, _L), jnp.float32),
    mesh=plsc.VectorSubcoreMesh(core_axis_name="c", subcore_axis_name="s",
                                num_cores=_NC, num_subcores=_NS),
    scratch_types=[
        pltpu.VMEM((_RPW, 2 * _N), jnp.float32),
        pltpu.VMEM((_L,), jnp.float32),
    ],
    compiler_params=pltpu.CompilerParams(needs_layout_passes=False),
)


@jax.jit
def kernel(yi, yi_t):
    comb, e2 = pl.pallas_call(
        _tc_dist_body,
        out_shape=[jax.ShapeDtypeStruct((_N, 2 * _N), jnp.float32),
                   jax.ShapeDtypeStruct((1, 1), jnp.float32)],
    )(yi, yi_t)
    parts = _sc_knn(comb)
    return jnp.sum(parts) + e2[0, 0]


# speedup vs baseline: 1.0771x; 1.0771x over previous
import jax
import jax.numpy as jnp
from jax import lax
from jax.experimental import pallas as pl
from jax.experimental.pallas import tpu as pltpu
from jax.experimental.pallas import tpu_sc as plsc

_T = 0.0025
_M = 0.6
_K = 16
_N = 256
_L = 16
_NC = 2
_NS = 16
_NW = _NC * _NS
_NTC = 0
_NSC = _N - _NTC
_RPW = _NSC // _NW
_NCHUNK = _N // _L


def _tc_dist_body(yi_ref, yit_ref, comb_ref, e2_ref):
    yi = yi_ref[...]
    yit = yit_ref[...]
    yin = yi * lax.rsqrt(jnp.sum(yi * yi, axis=1, keepdims=True) + 1e-12)
    yitn = yit * lax.rsqrt(jnp.sum(yit * yit, axis=1, keepdims=True) + 1e-12)
    g = lax.dot_general(yin, yin, (((1,), (1,)), ((), ())),
                        preferred_element_type=jnp.float32)
    c = lax.dot_general(yitn, yin, (((1,), (1,)), ((), ())),
                        preferred_element_type=jnp.float32)
    dii = 0.5 * jnp.sqrt(jnp.maximum(2.0 - 2.0 * g, 0.0) + 1e-12)
    dt = 0.5 * jnp.sqrt(jnp.maximum(2.0 - 2.0 * c, 0.0) + 1e-12)
    diag = (lax.broadcasted_iota(jnp.int32, (_N, _N), 0)
            == lax.broadcasted_iota(jnp.int32, (_N, _N), 1))
    dii_s = jnp.where(diag, 2.0, dii)
    comb_ref[:, : _N] = dii_s
    comb_ref[:, _N:] = dt
    d1 = jnp.min(dii_s, axis=1)
    diag_t = jnp.sum(jnp.where(diag, dt, 0.0), axis=1)
    e2 = jnp.sum(jnp.maximum(diag_t + _M - d1, 0.0))
    e2_ref[...] = jnp.reshape(e2 - _T * (_K * _N), (1, 1))


def _sc_body(comb_hbm, out_hbm, comb_v, acc_v):
    cid = lax.axis_index("c")
    sid = lax.axis_index("s")
    wid = sid * _NC + cid
    base = wid * _RPW
    pltpu.sync_copy(comb_hbm.at[pl.ds(base, _RPW)], comb_v)
    total = jnp.zeros((_L,), jnp.float32)
    for r in range(_RPW):
        bk = bv = None
        for c in range(_NCHUNK):
            key = comb_v[r, pl.ds(c * _L, _L)]
            val = comb_v[r, pl.ds(_N + c * _L, _L)]
            if c == 0:
                bk, bv = plsc.sort_key_val(key, val)
            else:
                ck, cv = plsc.sort_key_val(key, val, descending=True)
                keep = bk <= ck
                lk = jnp.where(keep, bk, ck)
                lv = jnp.where(keep, bv, cv)
                bk, bv = plsc.sort_key_val(lk, lv)
        diff = bk - bv
        total = total + diff * diff
    acc_v[...] = total
    pltpu.sync_copy(acc_v, out_hbm.at[wid])


_sc_knn = pl.kernel(
    _sc_body,
    out_type=jax.ShapeDtypeStruct((_NW, _L), jnp.float32),
    mesh=plsc.VectorSubcoreMesh(core_axis_name="c", subcore_axis_name="s",
                                num_cores=_NC, num_subcores=_NS),
    scratch_types=[
        pltpu.VMEM((_RPW, 2 * _N), jnp.float32),
        pltpu.VMEM((_L,), jnp.float32),
    ],
    compiler_params=pltpu.CompilerParams(needs_layout_passes=False),
)


@jax.jit
def kernel(yi, yi_t):
    comb, e2 = pl.pallas_call(
        _tc_dist_body,
        out_shape=[jax.ShapeDtypeStruct((_N, 2 * _N), jnp.float32),
                   jax.ShapeDtypeStruct((1, 1), jnp.float32)],
    )(yi, yi_t)
    parts = _sc_knn(comb)
    return jnp.sum(parts) + e2[0, 0]
